# baseline (speedup 1.0000x reference)
import jax
import jax.numpy as jnp
from jax.experimental import pallas as pl

N = 32768
K = 1024
D = 64
BETA = 0.25
BLOCK = 512


def _vq_block(z_ref, c_ref, z2_ref, c2_ref, zq_st_ref, zq_ref, idx_ref, loss_ref):
    z = z_ref[...]
    c = c_ref[...]
    z2 = z2_ref[...]
    c2 = c2_ref[...]

    zc = jax.lax.dot_general(
        z, c, dimension_numbers=(((1,), (1,)), ((), ())),
        preferred_element_type=jnp.float32)
    dists = z2 - 2.0 * zc + c2

    m = jnp.min(dists, axis=1, keepdims=True)
    iota = jax.lax.broadcasted_iota(jnp.int32, (BLOCK, K), 1)
    idx = jnp.min(jnp.where(dists == m, iota, jnp.int32(K)), axis=1)
    idx_ref[...] = idx

    one_hot = (iota == idx[:, None]).astype(jnp.float32)
    z_q = jax.lax.dot_general(
        one_hot, c, dimension_numbers=(((1,), (0,)), ((), ())),
        precision=jax.lax.Precision.HIGHEST,
        preferred_element_type=jnp.float32)
    zq_ref[...] = z_q
    zq_st_ref[...] = z + (z_q - z)

    diff = z_q - z
    part = jnp.sum(diff * diff).reshape(1, 1)

    @pl.when(pl.program_id(0) == 0)
    def _init():
        loss_ref[...] = jnp.zeros((1, 1), jnp.float32)

    loss_ref[...] += part


@jax.jit
def kernel(z_e, codebook):
    z2 = jnp.sum(z_e ** 2, axis=1, keepdims=True)
    c2 = jnp.sum(codebook ** 2, axis=1)[None, :]
    grid = N // BLOCK
    z_q_st, z_q, indices, loss_sum = pl.pallas_call(
        _vq_block,
        grid=(grid,),
        in_specs=[
            pl.BlockSpec((BLOCK, D), lambda i: (i, 0)),
            pl.BlockSpec((K, D), lambda i: (0, 0)),
            pl.BlockSpec((BLOCK, 1), lambda i: (i, 0)),
            pl.BlockSpec((1, K), lambda i: (0, 0)),
        ],
        out_specs=[
            pl.BlockSpec((BLOCK, D), lambda i: (i, 0)),
            pl.BlockSpec((BLOCK, D), lambda i: (i, 0)),
            pl.BlockSpec((BLOCK,), lambda i: (i,)),
            pl.BlockSpec((1, 1), lambda i: (0, 0)),
        ],
        out_shape=[
            jax.ShapeDtypeStruct((N, D), jnp.float32),
            jax.ShapeDtypeStruct((N, D), jnp.float32),
            jax.ShapeDtypeStruct((N,), jnp.int32),
            jax.ShapeDtypeStruct((1, 1), jnp.float32),
        ],
    )(z_e, codebook, z2, c2)
    m = loss_sum[0, 0] / float(N * D)
    loss_vq = m + BETA * m
    return (z_q_st, z_q, indices, loss_vq)


# speedup vs baseline: 1.5509x; 1.5509x over previous
import jax
import jax.numpy as jnp
from jax.experimental import pallas as pl

N = 32768
K = 1024
D = 64
BETA = 0.25
BLOCK = 512


def _vq_block(z_ref, c_ref, z2_ref, c2_ref, zq_st_ref, zq_ref, idx_ref, loss_ref):
    z = z_ref[...]
    c = c_ref[...]
    z2 = z2_ref[...]
    c2 = c2_ref[...]

    zc = jax.lax.dot_general(
        z, c, dimension_numbers=(((1,), (1,)), ((), ())),
        preferred_element_type=jnp.float32)
    dists = z2 - 2.0 * zc + c2

    m = jnp.min(dists, axis=1, keepdims=True)
    iota = jax.lax.broadcasted_iota(jnp.int32, (BLOCK, K), 1)
    idx = jnp.min(jnp.where(dists == m, iota, jnp.int32(K)), axis=1)
    idx_ref[...] = idx

    one_hot = (iota == idx[:, None]).astype(jnp.float32)
    z_q = jax.lax.dot_general(
        one_hot, c, dimension_numbers=(((1,), (0,)), ((), ())),
        preferred_element_type=jnp.float32)
    zq_ref[...] = z_q
    zq_st_ref[...] = z + (z_q - z)

    diff = z_q - z
    part = jnp.sum(diff * diff).reshape(1, 1)

    @pl.when(pl.program_id(0) == 0)
    def _init():
        loss_ref[...] = jnp.zeros((1, 1), jnp.float32)

    loss_ref[...] += part


@jax.jit
def kernel(z_e, codebook):
    z2 = jnp.sum(z_e ** 2, axis=1, keepdims=True)
    c2 = jnp.sum(codebook ** 2, axis=1)[None, :]
    grid = N // BLOCK
    z_q_st, z_q, indices, loss_sum = pl.pallas_call(
        _vq_block,
        grid=(grid,),
        in_specs=[
            pl.BlockSpec((BLOCK, D), lambda i: (i, 0)),
            pl.BlockSpec((K, D), lambda i: (0, 0)),
            pl.BlockSpec((BLOCK, 1), lambda i: (i, 0)),
            pl.BlockSpec((1, K), lambda i: (0, 0)),
        ],
        out_specs=[
            pl.BlockSpec((BLOCK, D), lambda i: (i, 0)),
            pl.BlockSpec((BLOCK, D), lambda i: (i, 0)),
            pl.BlockSpec((BLOCK,), lambda i: (i,)),
            pl.BlockSpec((1, 1), lambda i: (0, 0)),
        ],
        out_shape=[
            jax.ShapeDtypeStruct((N, D), jnp.float32),
            jax.ShapeDtypeStruct((N, D), jnp.float32),
            jax.ShapeDtypeStruct((N,), jnp.int32),
            jax.ShapeDtypeStruct((1, 1), jnp.float32),
        ],
    )(z_e, codebook, z2, c2)
    m = loss_sum[0, 0] / float(N * D)
    loss_vq = m + BETA * m
    return (z_q_st, z_q, indices, loss_vq)
